# baseline (speedup 1.0000x reference)
import functools

import jax
import jax.numpy as jnp
import numpy as np
from jax import lax
from jax.experimental import pallas as pl
from jax.experimental.pallas import tpu as pltpu
from jax.experimental.pallas import tpu_sc as plsc

C, H, W = 9, 260, 346
N = 4_000_000
NV = C * H * W
GRID_PAD = 811_008
SENT = NV
NC, NS = 2, 16
NW = NC * NS
ET = N // NW
EV_CHUNK = 7680
FULL_CHUNKS = 16
TAIL = ET - FULL_CHUNKS * EV_CHUNK
TAIL_ROWS = (TAIL + 127) // 128
PER_TILE_GRID = GRID_PAD // NS

TPT = N // NS
APB = 3 * EV_CHUNK
A_FULL = TPT // APB
A_TAIL = TPT - A_FULL * APB

_WH = np.float32(W * H)
_Wf = np.float32(W)
_CJ = [np.float32(j / C) for j in range(1, C)]


def _merge_body(a_ref, o_ref):
    o_ref[...] = a_ref[0] + a_ref[1]


def _sc_max_body(t_hbm, out_hbm, max_sh, buf_v):
    c_ax = lax.axis_index("c")
    s_ax = lax.axis_index("s")
    neg_inf = jnp.full((16,), -jnp.inf, jnp.float32)
    a_base = s_ax * TPT
    accs = (neg_inf, neg_inf, neg_inf, neg_inf)
    for k in range(A_FULL + 1):
        ln = APB if k < A_FULL else A_TAIL
        pltpu.sync_copy(t_hbm.at[pl.ds(a_base + k * APB, ln)],
                        buf_v.at[pl.ds(0, ln)])

        def _abody(i, a):
            o = i * 64
            return (jnp.maximum(a[0], buf_v[pl.ds(o, 16)]),
                    jnp.maximum(a[1], buf_v[pl.ds(o + 16, 16)]),
                    jnp.maximum(a[2], buf_v[pl.ds(o + 32, 16)]),
                    jnp.maximum(a[3], buf_v[pl.ds(o + 48, 16)]))

        accs = lax.fori_loop(0, ln // 64, _abody, accs)
        for r in range(ln // 64 * 64, ln, 16):
            accs = (jnp.maximum(accs[0], buf_v[pl.ds(r, 16)]),) + accs[1:]
    acc = jnp.maximum(jnp.maximum(accs[0], accs[1]),
                      jnp.maximum(accs[2], accs[3]))
    buf_v[pl.ds(0, 16)] = acc
    pltpu.sync_copy(buf_v.at[pl.ds(0, 16)], max_sh.at[pl.ds(s_ax * 16, 16)])
    plsc.subcore_barrier()

    @pl.when(s_ax == 0)
    def _():
        pltpu.sync_copy(max_sh, buf_v.at[pl.ds(0, NS * 16)])
        a = buf_v[pl.ds(0, 16)]
        for s in range(1, NS):
            a = jnp.maximum(a, buf_v[pl.ds(s * 16, 16)])
        buf_v[pl.ds(0, 16)] = a
        pltpu.sync_copy(buf_v.at[pl.ds(0, 16)], out_hbm.at[c_ax, pl.ds(0, 16)])


def _sc_body(x_hbm, y_hbm, t_hbm, tmax_hbm, out_hbm, grid_sh, ev_v, idx0_v,
             idx1_v, ones_v, zsrc_v, sem0, sem1, ssem0, ssem1):
    idx_bufs = (idx0_v, idx1_v)
    c_ax = lax.axis_index("c")
    s_ax = lax.axis_index("s")
    wid = c_ax * NS + s_ax
    lane = lax.iota(jnp.int32, 16)
    ev_base = wid * ET

    zeros16 = jnp.zeros((16,), jnp.float32)
    ones16 = jnp.ones((16,), jnp.float32)

    def _zbody(i, _):
        zsrc_v[pl.ds(i * 16, 16)] = zeros16
        ones_v[pl.ds(i * 16, 16)] = ones16
        return ()

    lax.fori_loop(0, EV_CHUNK // 16, _zbody, ())
    zoff = s_ax * PER_TILE_GRID
    ZREST = PER_TILE_GRID % EV_CHUNK

    def zero_copies():
        out = []
        for k in range(PER_TILE_GRID // EV_CHUNK):
            out.append((zsrc_v,
                        grid_sh.at[pl.ds(zoff + k * EV_CHUNK, EV_CHUNK)],
                        ssem0))
        if ZREST:
            out.append((
                zsrc_v.at[pl.ds(0, ZREST)],
                grid_sh.at[pl.ds(
                    zoff + (PER_TILE_GRID // EV_CHUNK) * EV_CHUNK, ZREST)],
                ssem0))
        return out

    for zc in zero_copies():
        pltpu.async_copy(*zc)

    def col_copies(cc, par, ln):
        off = ev_base + cc * EV_CHUNK
        boff = par * (3 * EV_CHUNK)
        sem = sem0 if par == 0 else sem1
        return [
            (x_hbm.at[pl.ds(off, ln)], ev_v.at[pl.ds(boff, ln)], sem),
            (y_hbm.at[pl.ds(off, ln)],
             ev_v.at[pl.ds(boff + EV_CHUNK, ln)], sem),
            (t_hbm.at[pl.ds(off, ln)],
             ev_v.at[pl.ds(boff + 2 * EV_CHUNK, ln)], sem),
        ]

    def start_chunk(cc, ln):
        for par in (0, 1):
            @pl.when(lax.rem(cc, 2) == par)
            def _():
                for c3 in col_copies(cc, par, ln):
                    pltpu.async_copy(*c3)

    def wait_chunk(cc, ln):
        for par in (0, 1):
            @pl.when(lax.rem(cc, 2) == par)
            def _():
                for c3 in col_copies(cc, par, ln):
                    pltpu.make_async_copy(*c3).wait()

    start_chunk(0, EV_CHUNK)

    pltpu.sync_copy(tmax_hbm.at[0, pl.ds(0, 16)], ev_v.at[pl.ds(APB, 16)])
    tmaxvec = jnp.broadcast_to(jnp.max(ev_v[pl.ds(APB, 16)]), (16,))

    for zc in zero_copies():
        pltpu.make_async_copy(*zc).wait()
    plsc.subcore_barrier()

    def compute16(boff, o):
        xv = ev_v[pl.ds(boff + o, 16)]
        yv = ev_v[pl.ds(boff + EV_CHUNK + o, 16)]
        tv = ev_v[pl.ds(boff + 2 * EV_CHUNK + o, 16)]
        tn = tv / tmaxvec
        base = jnp.where(tn > _CJ[0], _WH, np.float32(0.0))
        for j in range(1, 8):
            base = base + jnp.where(tn > _CJ[j], _WH, np.float32(0.0))
        s = (xv + _Wf * yv) + base
        idx = s.astype(jnp.int32)
        valid = jnp.logical_and(tn > np.float32(0.0), idx < NV)
        return jnp.where(valid, idx, SENT)

    def chunk_compute(par):
        boff = par * (3 * EV_CHUNK)
        idxb = idx_bufs[par]

        def qbody(q, _):
            o = q * 128
            for m in range(8):
                idxb[pl.ds(q * 128 + m * 16, 16)] = compute16(boff, o + m * 16)
            return ()

        lax.fori_loop(0, EV_CHUNK // 128, qbody, ())

    def tail_compute(par):
        boff = par * (3 * EV_CHUNK)
        idxb = idx_bufs[par]

        def qbody(q, _):
            o = q * 128
            for m in range(8):
                vec = compute16(boff, o + m * 16)
                eid = o + m * 16 + lane
                idxb[pl.ds(q * 128 + m * 16, 16)] = jnp.where(
                    eid < TAIL, vec, SENT)
            return ()

        lax.fori_loop(0, TAIL_ROWS, qbody, ())
        sent16 = jnp.full((16,), SENT, jnp.int32)

        def pbody(i, _):
            idxb[pl.ds(TAIL_ROWS * 128 + i * 16, 16)] = sent16
            return ()

        lax.fori_loop(0, (EV_CHUNK - TAIL_ROWS * 128) // 16, pbody, ())

    def scatter_copy(par):
        return (ones_v, grid_sh.at[idx_bufs[par]],
                (ssem0 if par == 0 else ssem1))

    def start_scatter(cc):
        for par in (0, 1):
            @pl.when(lax.rem(cc, 2) == par)
            def _():
                s, d, sem = scatter_copy(par)
                pltpu.async_copy(s, d, sem, add=True)

    def wait_scatter(cc):
        for par in (0, 1):
            @pl.when(lax.rem(cc, 2) == par)
            def _():
                s, d, sem = scatter_copy(par)
                pltpu.make_async_copy(s, d, sem).wait()

    def cbody(c, _):
        @pl.when(c < FULL_CHUNKS - 1)
        def _():
            start_chunk(c + 1, EV_CHUNK)

        @pl.when(c == FULL_CHUNKS - 1)
        def _():
            start_chunk(FULL_CHUNKS, TAIL)

        wait_chunk(c, EV_CHUNK)
        @pl.when(c >= 2)
        def _():
            wait_scatter(c)

        for par in (0, 1):
            @pl.when(lax.rem(c, 2) == par)
            def _():
                chunk_compute(par)

        start_scatter(c)
        return ()

    lax.fori_loop(0, FULL_CHUNKS, cbody, ())

    wait_chunk(FULL_CHUNKS, TAIL)
    wait_scatter(FULL_CHUNKS)
    tail_compute(FULL_CHUNKS % 2)
    start_scatter(FULL_CHUNKS)
    wait_scatter(FULL_CHUNKS)
    wait_scatter(FULL_CHUNKS + 1)

    plsc.subcore_barrier()
    ooff = s_ax * PER_TILE_GRID
    pltpu.sync_copy(grid_sh.at[pl.ds(ooff, PER_TILE_GRID)],
                    out_hbm.at[c_ax, pl.ds(ooff, PER_TILE_GRID)])


def _make_sc_max_call():
    mesh = plsc.VectorSubcoreMesh(core_axis_name="c", subcore_axis_name="s",
                                  num_cores=NC, num_subcores=NS)
    return pl.kernel(
        _sc_max_body,
        out_type=jax.ShapeDtypeStruct((NC, 128), jnp.float32),
        mesh=mesh,
        compiler_params=pltpu.CompilerParams(needs_layout_passes=False),
        scratch_types=[
            pltpu.VMEM_SHARED((NS * 16,), jnp.float32),
            pltpu.VMEM((APB,), jnp.float32),
        ],
    )


def _make_sc_call():
    mesh = plsc.VectorSubcoreMesh(core_axis_name="c", subcore_axis_name="s",
                                  num_cores=NC, num_subcores=NS)
    return pl.kernel(
        _sc_body,
        out_type=jax.ShapeDtypeStruct((NC, GRID_PAD), jnp.float32),
        mesh=mesh,
        compiler_params=pltpu.CompilerParams(needs_layout_passes=False),
        scratch_types=[
            pltpu.VMEM_SHARED((GRID_PAD,), jnp.float32),
            pltpu.VMEM((2 * 3 * EV_CHUNK,), jnp.float32),
            pltpu.VMEM((EV_CHUNK,), jnp.int32),
            pltpu.VMEM((EV_CHUNK,), jnp.int32),
            pltpu.VMEM((EV_CHUNK,), jnp.float32),
            pltpu.VMEM((EV_CHUNK,), jnp.float32),
            pltpu.SemaphoreType.DMA,
            pltpu.SemaphoreType.DMA,
            pltpu.SemaphoreType.DMA,
            pltpu.SemaphoreType.DMA,
        ],
    )


@jax.jit
def kernel(events):
    t = events[:, 2]
    tmaxp = _make_sc_max_call()(t)
    ev2 = lax.optimization_barrier(events)
    x = ev2[:, 0]
    y = ev2[:, 1]
    partials = _make_sc_call()(x, y, t, tmaxp)

    p3 = partials.reshape(NC, GRID_PAD // 128, 128)
    merged = pl.pallas_call(
        _merge_body,
        grid=(8,),
        in_specs=[pl.BlockSpec((NC, GRID_PAD // 128 // 8, 128),
                               lambda i: (0, i, 0))],
        out_specs=pl.BlockSpec((GRID_PAD // 128 // 8, 128), lambda i: (i, 0)),
        out_shape=jax.ShapeDtypeStruct((GRID_PAD // 128, 128), jnp.float32),
    )(p3)
    return merged.reshape(-1)[:NV].reshape(1, C, H, W)


# speedup vs baseline: 1.3270x; 1.3270x over previous
import functools

import jax
import jax.numpy as jnp
import numpy as np
from jax import lax
from jax.experimental import pallas as pl
from jax.experimental.pallas import tpu as pltpu
from jax.experimental.pallas import tpu_sc as plsc

C, H, W = 9, 260, 346
N = 4_000_000
NV = C * H * W
GRID_PAD = 811_008
SENT = NV
NC, NS = 2, 16
NW = NC * NS
ET = N // NW
EV_CHUNK = 7680
FULL_CHUNKS = 16
TAIL = ET - FULL_CHUNKS * EV_CHUNK
TAIL_ROWS = (TAIL + 127) // 128
PER_TILE_GRID = GRID_PAD // NS

TPT = N // NS
APB = 3 * EV_CHUNK
A_FULL = TPT // APB
A_TAIL = TPT - A_FULL * APB

_WH = np.float32(W * H)
_Wf = np.float32(W)
_CJ = [np.float32(j / C) for j in range(1, C)]


def _merge_body(a_ref, o_ref):
    o_ref[...] = a_ref[0] + a_ref[1]


def _sc_body(x_hbm, y_hbm, t_hbm, out_hbm, grid_sh, max_sh, ev_v, idx0_v,
             idx1_v, ones_v, sem0, sem1, ssem0, ssem1):
    idx_bufs = (idx0_v, idx1_v)
    c_ax = lax.axis_index("c")
    s_ax = lax.axis_index("s")
    wid = c_ax * NS + s_ax
    lane = lax.iota(jnp.int32, 16)
    ev_base = wid * ET

    zeros16 = jnp.zeros((16,), jnp.float32)

    def _zbody(i, _):
        ones_v[pl.ds(i * 16, 16)] = zeros16
        return ()

    lax.fori_loop(0, EV_CHUNK // 16, _zbody, ())
    zoff = s_ax * PER_TILE_GRID
    ZREST = PER_TILE_GRID % EV_CHUNK

    def zero_copies():
        out = []
        for k in range(PER_TILE_GRID // EV_CHUNK):
            out.append((ones_v,
                        grid_sh.at[pl.ds(zoff + k * EV_CHUNK, EV_CHUNK)],
                        ssem0))
        if ZREST:
            out.append((
                ones_v.at[pl.ds(0, ZREST)],
                grid_sh.at[pl.ds(
                    zoff + (PER_TILE_GRID // EV_CHUNK) * EV_CHUNK, ZREST)],
                ssem0))
        return out

    for zc in zero_copies():
        pltpu.async_copy(*zc)

    def col_copies(cc, par, ln):
        off = ev_base + cc * EV_CHUNK
        boff = par * (3 * EV_CHUNK)
        sem = sem0 if par == 0 else sem1
        return [
            (x_hbm.at[pl.ds(off, ln)], ev_v.at[pl.ds(boff, ln)], sem),
            (y_hbm.at[pl.ds(off, ln)],
             ev_v.at[pl.ds(boff + EV_CHUNK, ln)], sem),
            (t_hbm.at[pl.ds(off, ln)],
             ev_v.at[pl.ds(boff + 2 * EV_CHUNK, ln)], sem),
        ]

    def start_chunk(cc, ln):
        for par in (0, 1):
            @pl.when(lax.rem(cc, 2) == par)
            def _():
                for c3 in col_copies(cc, par, ln):
                    pltpu.async_copy(*c3)

    def wait_chunk(cc, ln):
        for par in (0, 1):
            @pl.when(lax.rem(cc, 2) == par)
            def _():
                for c3 in col_copies(cc, par, ln):
                    pltpu.make_async_copy(*c3).wait()

    start_chunk(0, EV_CHUNK)

    neg_inf = jnp.full((16,), -jnp.inf, jnp.float32)
    a_base = s_ax * TPT
    APB2 = APB // 2
    A2_FULL = TPT // APB2
    A2_TAIL = TPT - A2_FULL * APB2

    def a_copy(k):
        ln = APB2 if k < A2_FULL else A2_TAIL
        sub = k % 2
        return (t_hbm.at[pl.ds(a_base + k * APB2, ln)],
                ev_v.at[pl.ds(APB + sub * APB2, ln)],
                sem1 if sub == 0 else ssem1)

    pltpu.async_copy(*a_copy(0))
    accs = (neg_inf, neg_inf, neg_inf, neg_inf)
    for k in range(A2_FULL + 1):
        if k < A2_FULL:
            pltpu.async_copy(*a_copy(k + 1))
        pltpu.make_async_copy(*a_copy(k)).wait()
        ln = APB2 if k < A2_FULL else A2_TAIL
        o0 = APB + (k % 2) * APB2

        def _abody(i, a, o0=o0):
            o = o0 + i * 64
            return (jnp.maximum(a[0], ev_v[pl.ds(o, 16)]),
                    jnp.maximum(a[1], ev_v[pl.ds(o + 16, 16)]),
                    jnp.maximum(a[2], ev_v[pl.ds(o + 32, 16)]),
                    jnp.maximum(a[3], ev_v[pl.ds(o + 48, 16)]))

        accs = lax.fori_loop(0, ln // 64, _abody, accs)
        for r in range(ln // 64 * 64, ln, 16):
            accs = (jnp.maximum(accs[0], ev_v[pl.ds(o0 + r, 16)]),) + accs[1:]
    acc = jnp.maximum(jnp.maximum(accs[0], accs[1]),
                      jnp.maximum(accs[2], accs[3]))
    ev_v[pl.ds(APB, 16)] = acc
    pltpu.sync_copy(ev_v.at[pl.ds(APB, 16)], max_sh.at[pl.ds(s_ax * 16, 16)])
    for zc in zero_copies():
        pltpu.make_async_copy(*zc).wait()
    plsc.subcore_barrier()
    pltpu.sync_copy(max_sh, ev_v.at[pl.ds(APB, NS * 16)])
    acc = ev_v[pl.ds(APB, 16)]
    for s in range(1, NS):
        acc = jnp.maximum(acc, ev_v[pl.ds(APB + s * 16, 16)])
    tmaxvec = jnp.broadcast_to(jnp.max(acc), (16,))

    def _obody(i, _):
        ones_v[pl.ds(i * 16, 16)] = jnp.ones((16,), jnp.float32)
        return ()

    lax.fori_loop(0, EV_CHUNK // 16, _obody, ())

    plsc.subcore_barrier()

    def compute16(boff, o):
        xv = ev_v[pl.ds(boff + o, 16)]
        yv = ev_v[pl.ds(boff + EV_CHUNK + o, 16)]
        tv = ev_v[pl.ds(boff + 2 * EV_CHUNK + o, 16)]
        tn = tv / tmaxvec
        base = jnp.where(tn > _CJ[0], _WH, np.float32(0.0))
        for j in range(1, 8):
            base = base + jnp.where(tn > _CJ[j], _WH, np.float32(0.0))
        s = (xv + _Wf * yv) + base
        idx = s.astype(jnp.int32)
        valid = jnp.logical_and(tn > np.float32(0.0), idx < NV)
        return jnp.where(valid, idx, SENT)

    def chunk_compute(par):
        boff = par * (3 * EV_CHUNK)
        idxb = idx_bufs[par]

        def qbody(q, _):
            o = q * 128
            for m in range(8):
                idxb[pl.ds(q * 128 + m * 16, 16)] = compute16(boff, o + m * 16)
            return ()

        lax.fori_loop(0, EV_CHUNK // 128, qbody, ())

    def tail_compute(par):
        boff = par * (3 * EV_CHUNK)
        idxb = idx_bufs[par]

        def qbody(q, _):
            o = q * 128
            for m in range(8):
                vec = compute16(boff, o + m * 16)
                eid = o + m * 16 + lane
                idxb[pl.ds(q * 128 + m * 16, 16)] = jnp.where(
                    eid < TAIL, vec, SENT)
            return ()

        lax.fori_loop(0, TAIL_ROWS, qbody, ())
        sent16 = jnp.full((16,), SENT, jnp.int32)

        def pbody(i, _):
            idxb[pl.ds(TAIL_ROWS * 128 + i * 16, 16)] = sent16
            return ()

        lax.fori_loop(0, (EV_CHUNK - TAIL_ROWS * 128) // 16, pbody, ())

    def scatter_copy(par):
        return (ones_v, grid_sh.at[idx_bufs[par]],
                (ssem0 if par == 0 else ssem1))

    def start_scatter(cc):
        for par in (0, 1):
            @pl.when(lax.rem(cc, 2) == par)
            def _():
                s, d, sem = scatter_copy(par)
                pltpu.async_copy(s, d, sem, add=True)

    def wait_scatter(cc):
        for par in (0, 1):
            @pl.when(lax.rem(cc, 2) == par)
            def _():
                s, d, sem = scatter_copy(par)
                pltpu.make_async_copy(s, d, sem).wait()

    def cbody(c, _):
        @pl.when(c < FULL_CHUNKS - 1)
        def _():
            start_chunk(c + 1, EV_CHUNK)

        @pl.when(c == FULL_CHUNKS - 1)
        def _():
            start_chunk(FULL_CHUNKS, TAIL)

        wait_chunk(c, EV_CHUNK)
        @pl.when(c >= 2)
        def _():
            wait_scatter(c)

        for par in (0, 1):
            @pl.when(lax.rem(c, 2) == par)
            def _():
                chunk_compute(par)

        start_scatter(c)
        return ()

    lax.fori_loop(0, FULL_CHUNKS, cbody, ())

    wait_chunk(FULL_CHUNKS, TAIL)
    wait_scatter(FULL_CHUNKS)
    tail_compute(FULL_CHUNKS % 2)
    start_scatter(FULL_CHUNKS)
    wait_scatter(FULL_CHUNKS)
    wait_scatter(FULL_CHUNKS + 1)

    plsc.subcore_barrier()
    ooff = s_ax * PER_TILE_GRID
    pltpu.sync_copy(grid_sh.at[pl.ds(ooff, PER_TILE_GRID)],
                    out_hbm.at[c_ax, pl.ds(ooff, PER_TILE_GRID)])


def _make_sc_call():
    mesh = plsc.VectorSubcoreMesh(core_axis_name="c", subcore_axis_name="s",
                                  num_cores=NC, num_subcores=NS)
    return pl.kernel(
        _sc_body,
        out_type=jax.ShapeDtypeStruct((NC, GRID_PAD), jnp.float32),
        mesh=mesh,
        compiler_params=pltpu.CompilerParams(needs_layout_passes=False),
        scratch_types=[
            pltpu.VMEM_SHARED((GRID_PAD,), jnp.float32),
            pltpu.VMEM_SHARED((NS * 16,), jnp.float32),
            pltpu.VMEM((2 * 3 * EV_CHUNK,), jnp.float32),
            pltpu.VMEM((EV_CHUNK,), jnp.int32),
            pltpu.VMEM((EV_CHUNK,), jnp.int32),
            pltpu.VMEM((EV_CHUNK,), jnp.float32),
            pltpu.SemaphoreType.DMA,
            pltpu.SemaphoreType.DMA,
            pltpu.SemaphoreType.DMA,
            pltpu.SemaphoreType.DMA,
        ],
    )


@jax.jit
def kernel(events):
    x = events[:, 0]
    y = events[:, 1]
    t = events[:, 2]
    partials = _make_sc_call()(x, y, t)

    p3 = partials.reshape(NC, GRID_PAD // 128, 128)
    merged = pl.pallas_call(
        _merge_body,
        grid=(8,),
        in_specs=[pl.BlockSpec((NC, GRID_PAD // 128 // 8, 128),
                               lambda i: (0, i, 0))],
        out_specs=pl.BlockSpec((GRID_PAD // 128 // 8, 128), lambda i: (i, 0)),
        out_shape=jax.ShapeDtypeStruct((GRID_PAD // 128, 128), jnp.float32),
    )(p3)
    return merged.reshape(-1)[:NV].reshape(1, C, H, W)


# speedup vs baseline: 1.7195x; 1.2958x over previous
import functools

import jax
import jax.numpy as jnp
import numpy as np
from jax import lax
from jax.experimental import pallas as pl
from jax.experimental.pallas import tpu as pltpu
from jax.experimental.pallas import tpu_sc as plsc

C, H, W = 9, 260, 346
N = 4_000_000
NV = C * H * W
GRID_PAD = 811_008
SENT = NV
NC, NS = 2, 16
NW = NC * NS
ET = N // NW
EV_CHUNK = 7680
FULL_CHUNKS = 16
TAIL = ET - FULL_CHUNKS * EV_CHUNK
TAIL_ROWS = (TAIL + 127) // 128
PER_TILE_GRID = GRID_PAD // NS

TPT = N // NS
APB = 3 * EV_CHUNK
A_FULL = TPT // APB
A_TAIL = TPT - A_FULL * APB

_WH = np.float32(W * H)
_Wf = np.float32(W)
_CJ = [np.float32(j / C) for j in range(1, C)]


def _merge_body(a_ref, o_ref):
    o_ref[...] = a_ref[0] + a_ref[1]


def _sc_body(x_hbm, y_hbm, t_hbm, out_hbm, grid_sh, max_sh, ev_v, idx0_v,
             idx1_v, ones_v, tidx_v, tones_v, sem0, sem1, ssem0, ssem1):
    idx_bufs = (idx0_v, idx1_v)
    c_ax = lax.axis_index("c")
    s_ax = lax.axis_index("s")
    wid = c_ax * NS + s_ax
    lane = lax.iota(jnp.int32, 16)
    ev_base = wid * ET

    zeros16 = jnp.zeros((16,), jnp.float32)

    def _zbody(i, _):
        ones_v[pl.ds(i * 16, 16)] = zeros16
        return ()

    lax.fori_loop(0, EV_CHUNK // 16, _zbody, ())
    zoff = s_ax * PER_TILE_GRID
    ZREST = PER_TILE_GRID % EV_CHUNK

    def zero_copies():
        out = []
        for k in range(PER_TILE_GRID // EV_CHUNK):
            out.append((ones_v,
                        grid_sh.at[pl.ds(zoff + k * EV_CHUNK, EV_CHUNK)],
                        ssem0))
        if ZREST:
            out.append((
                ones_v.at[pl.ds(0, ZREST)],
                grid_sh.at[pl.ds(
                    zoff + (PER_TILE_GRID // EV_CHUNK) * EV_CHUNK, ZREST)],
                ssem0))
        return out

    for zc in zero_copies():
        pltpu.async_copy(*zc)

    def col_copies(cc, par, ln):
        off = ev_base + cc * EV_CHUNK
        boff = par * (3 * EV_CHUNK)
        sem = sem0 if par == 0 else sem1
        return [
            (x_hbm.at[pl.ds(off, ln)], ev_v.at[pl.ds(boff, ln)], sem),
            (y_hbm.at[pl.ds(off, ln)],
             ev_v.at[pl.ds(boff + EV_CHUNK, ln)], sem),
            (t_hbm.at[pl.ds(off, ln)],
             ev_v.at[pl.ds(boff + 2 * EV_CHUNK, ln)], sem),
        ]

    def start_chunk(cc, ln):
        for par in (0, 1):
            @pl.when(lax.rem(cc, 2) == par)
            def _():
                for c3 in col_copies(cc, par, ln):
                    pltpu.async_copy(*c3)

    def wait_chunk(cc, ln):
        for par in (0, 1):
            @pl.when(lax.rem(cc, 2) == par)
            def _():
                for c3 in col_copies(cc, par, ln):
                    pltpu.make_async_copy(*c3).wait()

    start_chunk(0, EV_CHUNK)

    neg_inf = jnp.full((16,), -jnp.inf, jnp.float32)
    a_base = s_ax * TPT
    APB2 = APB // 2
    A2_FULL = TPT // APB2
    A2_TAIL = TPT - A2_FULL * APB2

    def a_copy(k):
        ln = APB2 if k < A2_FULL else A2_TAIL
        sub = k % 2
        return (t_hbm.at[pl.ds(a_base + k * APB2, ln)],
                ev_v.at[pl.ds(APB + sub * APB2, ln)],
                sem1 if sub == 0 else ssem1)

    pltpu.async_copy(*a_copy(0))
    accs = (neg_inf, neg_inf, neg_inf, neg_inf)
    for k in range(A2_FULL + 1):
        if k < A2_FULL:
            pltpu.async_copy(*a_copy(k + 1))
        pltpu.make_async_copy(*a_copy(k)).wait()
        ln = APB2 if k < A2_FULL else A2_TAIL
        o0 = APB + (k % 2) * APB2

        def _abody(i, a, o0=o0):
            o = o0 + i * 64
            return (jnp.maximum(a[0], ev_v[pl.ds(o, 16)]),
                    jnp.maximum(a[1], ev_v[pl.ds(o + 16, 16)]),
                    jnp.maximum(a[2], ev_v[pl.ds(o + 32, 16)]),
                    jnp.maximum(a[3], ev_v[pl.ds(o + 48, 16)]))

        accs = lax.fori_loop(0, ln // 64, _abody, accs)
        for r in range(ln // 64 * 64, ln, 16):
            accs = (jnp.maximum(accs[0], ev_v[pl.ds(o0 + r, 16)]),) + accs[1:]
    acc = jnp.maximum(jnp.maximum(accs[0], accs[1]),
                      jnp.maximum(accs[2], accs[3]))
    ev_v[pl.ds(APB, 16)] = acc
    pltpu.sync_copy(ev_v.at[pl.ds(APB, 16)], max_sh.at[pl.ds(s_ax * 16, 16)])
    for zc in zero_copies():
        pltpu.make_async_copy(*zc).wait()
    plsc.subcore_barrier()
    pltpu.sync_copy(max_sh, ev_v.at[pl.ds(APB, NS * 16)])
    acc = ev_v[pl.ds(APB, 16)]
    for s in range(1, NS):
        acc = jnp.maximum(acc, ev_v[pl.ds(APB + s * 16, 16)])
    tmaxvec = jnp.broadcast_to(jnp.max(acc), (16,))

    ones16 = jnp.ones((16,), jnp.float32)

    def _obody(i, _):
        ones_v[pl.ds(i * 16, 16)] = ones16
        return ()

    lax.fori_loop(0, EV_CHUNK // 16, _obody, ())

    def _tbody(i, _):
        tones_v[pl.ds(i * 16, 16)] = ones16
        return ()

    lax.fori_loop(0, TAIL_ROWS * 128 // 16, _tbody, ())

    plsc.subcore_barrier()

    def compute16(boff, o):
        xv = ev_v[pl.ds(boff + o, 16)]
        yv = ev_v[pl.ds(boff + EV_CHUNK + o, 16)]
        tv = ev_v[pl.ds(boff + 2 * EV_CHUNK + o, 16)]
        tn = tv / tmaxvec
        base = jnp.where(tn > _CJ[0], _WH, np.float32(0.0))
        for j in range(1, 8):
            base = base + jnp.where(tn > _CJ[j], _WH, np.float32(0.0))
        s = (xv + _Wf * yv) + base
        idx = s.astype(jnp.int32)
        valid = jnp.logical_and(tn > np.float32(0.0), idx < NV)
        return jnp.where(valid, idx, SENT)

    def chunk_compute(par):
        boff = par * (3 * EV_CHUNK)
        idxb = idx_bufs[par]

        def qbody(q, _):
            o = q * 128
            for m in range(8):
                idxb[pl.ds(q * 128 + m * 16, 16)] = compute16(boff, o + m * 16)
            return ()

        lax.fori_loop(0, EV_CHUNK // 128, qbody, ())

    def tail_compute(par):
        boff = par * (3 * EV_CHUNK)

        def qbody(q, _):
            o = q * 128
            for m in range(8):
                vec = compute16(boff, o + m * 16)
                eid = o + m * 16 + lane
                tidx_v[pl.ds(q * 128 + m * 16, 16)] = jnp.where(
                    eid < TAIL, vec, SENT)
            return ()

        lax.fori_loop(0, TAIL_ROWS, qbody, ())

    def scatter_copy(par):
        return (ones_v, grid_sh.at[idx_bufs[par]],
                (ssem0 if par == 0 else ssem1))

    def start_scatter(cc):
        for par in (0, 1):
            @pl.when(lax.rem(cc, 2) == par)
            def _():
                s, d, sem = scatter_copy(par)
                pltpu.async_copy(s, d, sem, add=True)

    def wait_scatter(cc):
        for par in (0, 1):
            @pl.when(lax.rem(cc, 2) == par)
            def _():
                s, d, sem = scatter_copy(par)
                pltpu.make_async_copy(s, d, sem).wait()

    def cbody(c, _):
        @pl.when(c < FULL_CHUNKS - 1)
        def _():
            start_chunk(c + 1, EV_CHUNK)

        @pl.when(c == FULL_CHUNKS - 1)
        def _():
            start_chunk(FULL_CHUNKS, TAIL)

        wait_chunk(c, EV_CHUNK)
        @pl.when(c >= 2)
        def _():
            wait_scatter(c)

        for par in (0, 1):
            @pl.when(lax.rem(c, 2) == par)
            def _():
                chunk_compute(par)

        start_scatter(c)
        return ()

    lax.fori_loop(0, FULL_CHUNKS, cbody, ())

    wait_chunk(FULL_CHUNKS, TAIL)
    tail_compute(FULL_CHUNKS % 2)
    pltpu.sync_copy(tones_v, grid_sh.at[tidx_v], add=True)
    wait_scatter(FULL_CHUNKS)
    wait_scatter(FULL_CHUNKS + 1)

    plsc.subcore_barrier()
    ooff = s_ax * PER_TILE_GRID
    pltpu.sync_copy(grid_sh.at[pl.ds(ooff, PER_TILE_GRID)],
                    out_hbm.at[c_ax, pl.ds(ooff, PER_TILE_GRID)])


def _make_sc_call():
    mesh = plsc.VectorSubcoreMesh(core_axis_name="c", subcore_axis_name="s",
                                  num_cores=NC, num_subcores=NS)
    return pl.kernel(
        _sc_body,
        out_type=jax.ShapeDtypeStruct((NC, GRID_PAD), jnp.float32),
        mesh=mesh,
        compiler_params=pltpu.CompilerParams(needs_layout_passes=False),
        scratch_types=[
            pltpu.VMEM_SHARED((GRID_PAD,), jnp.float32),
            pltpu.VMEM_SHARED((NS * 16,), jnp.float32),
            pltpu.VMEM((2 * 3 * EV_CHUNK,), jnp.float32),
            pltpu.VMEM((EV_CHUNK,), jnp.int32),
            pltpu.VMEM((EV_CHUNK,), jnp.int32),
            pltpu.VMEM((EV_CHUNK,), jnp.float32),
            pltpu.VMEM((TAIL_ROWS * 128,), jnp.int32),
            pltpu.VMEM((TAIL_ROWS * 128,), jnp.float32),
            pltpu.SemaphoreType.DMA,
            pltpu.SemaphoreType.DMA,
            pltpu.SemaphoreType.DMA,
            pltpu.SemaphoreType.DMA,
        ],
    )


@jax.jit
def kernel(events):
    x = events[:, 0]
    y = events[:, 1]
    t = events[:, 2]
    partials = _make_sc_call()(x, y, t)

    p3 = partials.reshape(NC, GRID_PAD // 128, 128)
    merged = pl.pallas_call(
        _merge_body,
        grid=(8,),
        in_specs=[pl.BlockSpec((NC, GRID_PAD // 128 // 8, 128),
                               lambda i: (0, i, 0))],
        out_specs=pl.BlockSpec((GRID_PAD // 128 // 8, 128), lambda i: (i, 0)),
        out_shape=jax.ShapeDtypeStruct((GRID_PAD // 128, 128), jnp.float32),
    )(p3)
    return merged.reshape(-1)[:NV].reshape(1, C, H, W)


# speedup vs baseline: 1.7268x; 1.0043x over previous
import jax
import jax.numpy as jnp
import numpy as np
from jax import lax
from jax.experimental import pallas as pl
from jax.experimental.pallas import tpu as pltpu
from jax.experimental.pallas import tpu_sc as plsc

C, H, W = 9, 260, 346
N = 4_000_000
NV = C * H * W
GRID_PAD = 811_008
SENT = NV
NC, NS = 2, 16
NW = NC * NS
ET = N // NW
EV_CHUNK = 7680
FULL_CHUNKS = 16
TAIL = ET - FULL_CHUNKS * EV_CHUNK
TAIL_ROWS = (TAIL + 127) // 128
PER_TILE_GRID = GRID_PAD // NS

TPT = N // NS
APB = 3 * EV_CHUNK
A_FULL = TPT // APB
A_TAIL = TPT - A_FULL * APB

_WH = np.float32(W * H)
_Wf = np.float32(W)
_CJ = [np.float32(j / C) for j in range(1, C)]


def _merge_body(a_ref, o_ref):
    o_ref[...] = a_ref[0] + a_ref[1]


def _sc_body(x_hbm, y_hbm, t_hbm, out_hbm, grid_sh, max_sh, ev_v, idx0_v,
             idx1_v, ones_v, tidx_v, tones_v, sem0, sem1, ssem0, ssem1):
    idx_bufs = (idx0_v, idx1_v)
    c_ax = lax.axis_index("c")
    s_ax = lax.axis_index("s")
    wid = c_ax * NS + s_ax
    lane = lax.iota(jnp.int32, 16)
    ev_base = wid * ET

    zeros16 = jnp.zeros((16,), jnp.float32)

    def _zbody(i, _):
        ones_v[pl.ds(i * 16, 16)] = zeros16
        return ()

    lax.fori_loop(0, EV_CHUNK // 16, _zbody, ())
    zoff = s_ax * PER_TILE_GRID
    ZREST = PER_TILE_GRID % EV_CHUNK

    def zero_copies():
        out = []
        for k in range(PER_TILE_GRID // EV_CHUNK):
            out.append((ones_v,
                        grid_sh.at[pl.ds(zoff + k * EV_CHUNK, EV_CHUNK)],
                        ssem0))
        if ZREST:
            out.append((
                ones_v.at[pl.ds(0, ZREST)],
                grid_sh.at[pl.ds(
                    zoff + (PER_TILE_GRID // EV_CHUNK) * EV_CHUNK, ZREST)],
                ssem0))
        return out

    for zc in zero_copies():
        pltpu.async_copy(*zc)

    def col_copies(cc, par, ln):
        off = ev_base + cc * EV_CHUNK
        boff = par * (3 * EV_CHUNK)
        sem = sem0 if par == 0 else sem1
        return [
            (x_hbm.at[pl.ds(off, ln)], ev_v.at[pl.ds(boff, ln)], sem),
            (y_hbm.at[pl.ds(off, ln)],
             ev_v.at[pl.ds(boff + EV_CHUNK, ln)], sem),
            (t_hbm.at[pl.ds(off, ln)],
             ev_v.at[pl.ds(boff + 2 * EV_CHUNK, ln)], sem),
        ]

    def start_chunk(cc, ln):
        for par in (0, 1):
            @pl.when(lax.rem(cc, 2) == par)
            def _():
                for c3 in col_copies(cc, par, ln):
                    pltpu.async_copy(*c3)

    def wait_chunk(cc, ln):
        for par in (0, 1):
            @pl.when(lax.rem(cc, 2) == par)
            def _():
                for c3 in col_copies(cc, par, ln):
                    pltpu.make_async_copy(*c3).wait()

    start_chunk(0, EV_CHUNK)

    neg_inf = jnp.full((16,), -jnp.inf, jnp.float32)
    a_base = s_ax * TPT
    APB2 = APB // 2
    A2_FULL = TPT // APB2
    A2_TAIL = TPT - A2_FULL * APB2

    def a_copy(k):
        ln = APB2 if k < A2_FULL else A2_TAIL
        sub = k % 2
        return (t_hbm.at[pl.ds(a_base + k * APB2, ln)],
                ev_v.at[pl.ds(APB + sub * APB2, ln)],
                sem1 if sub == 0 else ssem1)

    pltpu.async_copy(*a_copy(0))
    accs = (neg_inf, neg_inf, neg_inf, neg_inf)
    for k in range(A2_FULL + 1):
        if k < A2_FULL:
            pltpu.async_copy(*a_copy(k + 1))
        pltpu.make_async_copy(*a_copy(k)).wait()
        ln = APB2 if k < A2_FULL else A2_TAIL
        o0 = APB + (k % 2) * APB2

        def _abody(i, a, o0=o0):
            o = o0 + i * 64
            return (jnp.maximum(a[0], ev_v[pl.ds(o, 16)]),
                    jnp.maximum(a[1], ev_v[pl.ds(o + 16, 16)]),
                    jnp.maximum(a[2], ev_v[pl.ds(o + 32, 16)]),
                    jnp.maximum(a[3], ev_v[pl.ds(o + 48, 16)]))

        accs = lax.fori_loop(0, ln // 64, _abody, accs)
        for r in range(ln // 64 * 64, ln, 16):
            accs = (jnp.maximum(accs[0], ev_v[pl.ds(o0 + r, 16)]),) + accs[1:]
    acc = jnp.maximum(jnp.maximum(accs[0], accs[1]),
                      jnp.maximum(accs[2], accs[3]))
    ev_v[pl.ds(APB, 16)] = acc
    pltpu.sync_copy(ev_v.at[pl.ds(APB, 16)], max_sh.at[pl.ds(s_ax * 16, 16)])
    for zc in zero_copies():
        pltpu.make_async_copy(*zc).wait()
    plsc.subcore_barrier()
    pltpu.sync_copy(max_sh, ev_v.at[pl.ds(APB, NS * 16)])
    acc = ev_v[pl.ds(APB, 16)]
    for s in range(1, NS):
        acc = jnp.maximum(acc, ev_v[pl.ds(APB + s * 16, 16)])
    tmaxvec = jnp.broadcast_to(jnp.max(acc), (16,))

    ones16 = jnp.ones((16,), jnp.float32)

    def _obody(i, _):
        ones_v[pl.ds(i * 16, 16)] = ones16
        return ()

    lax.fori_loop(0, EV_CHUNK // 16, _obody, ())

    def _tbody(i, _):
        tones_v[pl.ds(i * 16, 16)] = ones16
        return ()

    lax.fori_loop(0, TAIL_ROWS * 128 // 16, _tbody, ())

    plsc.subcore_barrier()

    def compute16(boff, o):
        xv = ev_v[pl.ds(boff + o, 16)]
        yv = ev_v[pl.ds(boff + EV_CHUNK + o, 16)]
        tv = ev_v[pl.ds(boff + 2 * EV_CHUNK + o, 16)]
        tn = tv / tmaxvec
        base = jnp.where(tn > _CJ[0], _WH, np.float32(0.0))
        for j in range(1, 8):
            base = base + jnp.where(tn > _CJ[j], _WH, np.float32(0.0))
        s = (xv + _Wf * yv) + base
        idx = s.astype(jnp.int32)
        valid = jnp.logical_and(tn > np.float32(0.0), idx < NV)
        return jnp.where(valid, idx, SENT)

    def chunk_compute(par):
        boff = par * (3 * EV_CHUNK)
        idxb = idx_bufs[par]

        def qbody(q, _):
            o = q * 128
            for m in range(8):
                idxb[pl.ds(q * 128 + m * 16, 16)] = compute16(boff, o + m * 16)
            return ()

        lax.fori_loop(0, EV_CHUNK // 128, qbody, ())

    def tail_compute(par):
        boff = par * (3 * EV_CHUNK)

        def qbody(q, _):
            o = q * 128
            for m in range(8):
                vec = compute16(boff, o + m * 16)
                eid = o + m * 16 + lane
                tidx_v[pl.ds(q * 128 + m * 16, 16)] = jnp.where(
                    eid < TAIL, vec, SENT)
            return ()

        lax.fori_loop(0, TAIL_ROWS, qbody, ())

    def scatter_copy(par):
        return (ones_v, grid_sh.at[idx_bufs[par]],
                (ssem0 if par == 0 else ssem1))

    def start_scatter(cc):
        for par in (0, 1):
            @pl.when(lax.rem(cc, 2) == par)
            def _():
                s, d, sem = scatter_copy(par)
                pltpu.async_copy(s, d, sem, add=True)

    def wait_scatter(cc):
        for par in (0, 1):
            @pl.when(lax.rem(cc, 2) == par)
            def _():
                s, d, sem = scatter_copy(par)
                pltpu.make_async_copy(s, d, sem).wait()

    def cbody(c, _):
        @pl.when(c < FULL_CHUNKS - 1)
        def _():
            start_chunk(c + 1, EV_CHUNK)

        @pl.when(c == FULL_CHUNKS - 1)
        def _():
            start_chunk(FULL_CHUNKS, TAIL)

        wait_chunk(c, EV_CHUNK)
        @pl.when(c >= 2)
        def _():
            wait_scatter(c)

        for par in (0, 1):
            @pl.when(lax.rem(c, 2) == par)
            def _():
                chunk_compute(par)

        start_scatter(c)
        return ()

    lax.fori_loop(0, FULL_CHUNKS, cbody, ())

    wait_chunk(FULL_CHUNKS, TAIL)
    tail_compute(FULL_CHUNKS % 2)
    pltpu.sync_copy(tones_v, grid_sh.at[tidx_v], add=True)
    wait_scatter(FULL_CHUNKS)
    wait_scatter(FULL_CHUNKS + 1)

    plsc.subcore_barrier()
    ooff = s_ax * PER_TILE_GRID
    pltpu.sync_copy(grid_sh.at[pl.ds(ooff, PER_TILE_GRID)],
                    out_hbm.at[c_ax, pl.ds(ooff, PER_TILE_GRID)])


def _make_sc_call():
    mesh = plsc.VectorSubcoreMesh(core_axis_name="c", subcore_axis_name="s",
                                  num_cores=NC, num_subcores=NS)
    return pl.kernel(
        _sc_body,
        out_type=jax.ShapeDtypeStruct((NC, GRID_PAD), jnp.float32),
        mesh=mesh,
        compiler_params=pltpu.CompilerParams(needs_layout_passes=False),
        scratch_types=[
            pltpu.VMEM_SHARED((GRID_PAD,), jnp.float32),
            pltpu.VMEM_SHARED((NS * 16,), jnp.float32),
            pltpu.VMEM((2 * 3 * EV_CHUNK,), jnp.float32),
            pltpu.VMEM((EV_CHUNK,), jnp.int32),
            pltpu.VMEM((EV_CHUNK,), jnp.int32),
            pltpu.VMEM((EV_CHUNK,), jnp.float32),
            pltpu.VMEM((TAIL_ROWS * 128,), jnp.int32),
            pltpu.VMEM((TAIL_ROWS * 128,), jnp.float32),
            pltpu.SemaphoreType.DMA,
            pltpu.SemaphoreType.DMA,
            pltpu.SemaphoreType.DMA,
            pltpu.SemaphoreType.DMA,
        ],
    )


@jax.jit
def kernel(events):
    x = events[:, 0]
    y = events[:, 1]
    t = events[:, 2]
    partials = _make_sc_call()(x, y, t)

    p3 = partials.reshape(NC, GRID_PAD // 128, 128)
    merged = pl.pallas_call(
        _merge_body,
        grid=(8,),
        in_specs=[pl.BlockSpec((NC, GRID_PAD // 128 // 8, 128),
                               lambda i: (0, i, 0))],
        out_specs=pl.BlockSpec((GRID_PAD // 128 // 8, 128), lambda i: (i, 0)),
        out_shape=jax.ShapeDtypeStruct((GRID_PAD // 128, 128), jnp.float32),
    )(p3)
    return merged.reshape(-1)[:NV].reshape(1, C, H, W)
